# baseline (speedup 1.0000x reference)
import jax
import jax.numpy as jnp
from jax.experimental import pallas as pl
from jax.experimental.pallas import tpu as pltpu

_TILE = 32768
_CHUNK = 128


def _score_matrix_kernel(u_ref, i_ref, out_ref):
    out_ref[...] = jax.lax.dot_general(
        u_ref[...], i_ref[...], (((1,), (1,)), ((), ())),
        preferred_element_type=jnp.float32).astype(jnp.bfloat16)


def _gather_kernel(fb_ref, ilo_ref, s_ref, out_ref):
    ones = jnp.ones((8, 128), jnp.bfloat16)
    iota = jax.lax.broadcasted_iota(jnp.int32, (_CHUNK, 128), 1)
    ncol = _CHUNK // 8

    def gather(ci):
        base = ci * _CHUNK
        rows = []
        for j in range(_CHUNK):
            rows.append(s_ref[fb_ref[0, base + j]])
        return jnp.concatenate(rows, axis=0), ilo_ref[ci]

    def reduce_store(blk, win, ci):
        ev = pltpu.bitcast(blk << 16, jnp.float32)
        od = pltpu.bitcast(blk & jnp.int32(-65536), jnp.float32)
        l0f = jnp.concatenate(
            [jnp.broadcast_to(win[:, v:v + 1], (8, 128))
             for v in range(ncol)], axis=0)
        sel = jnp.where((l0f & 1) == 1, od, ev)
        blk2 = jnp.where(iota == (l0f >> 1), sel, 0.0).astype(jnp.bfloat16)
        acc = jax.lax.dot_general(ones, blk2, (((1,), (1,)), ((), ())),
                                  preferred_element_type=jnp.float32)
        off = pl.multiple_of(ci * _CHUNK, _CHUNK)
        out_ref[:, pl.ds(off, _CHUNK)] = acc[0:1, :]

    def body(ci, carry):
        blk_prev, win_prev = carry
        reduce_store(blk_prev, win_prev, ci - 1)
        return gather(ci)

    last = jax.lax.fori_loop(1, _TILE // _CHUNK, body, gather(0))
    reduce_store(last[0], last[1], _TILE // _CHUNK - 1)


def kernel(user_ids, item_ids, user_emb, item_emb, user_bias, item_bias):
    B = user_ids.shape[0]
    Nu, D = user_emb.shape
    Ni = item_emb.shape[0]
    Daug = D + 2

    u_aug = jnp.concatenate(
        [user_emb.astype(jnp.float32),
         user_bias.reshape(Nu, 1).astype(jnp.float32),
         jnp.ones((Nu, 1), jnp.float32)], axis=1)
    i_aug = jnp.concatenate(
        [item_emb.astype(jnp.float32),
         jnp.ones((Ni, 1), jnp.float32),
         item_bias.reshape(Ni, 1).astype(jnp.float32)], axis=1)

    Dpad = ((Daug + 127) // 128) * 128
    Nip = ((Ni + 255) // 256) * 256
    u_aug = jnp.pad(u_aug, ((0, 0), (0, Dpad - Daug)))
    i_aug = jnp.pad(i_aug, ((0, Nip - Ni), (0, Dpad - Daug)))
    u_bf = u_aug.astype(jnp.bfloat16)
    i_bf = i_aug.astype(jnp.bfloat16)

    ublk = min(512, Nu)
    smat = pl.pallas_call(
        _score_matrix_kernel,
        out_shape=jax.ShapeDtypeStruct((Nu, Nip), jnp.bfloat16),
        grid=(Nu // ublk,),
        in_specs=[
            pl.BlockSpec((ublk, Dpad), lambda i: (i, 0)),
            pl.BlockSpec((Nip, Dpad), lambda i: (0, 0)),
        ],
        out_specs=pl.BlockSpec((ublk, Nip), lambda i: (i, 0)),
        compiler_params=pltpu.CompilerParams(
            dimension_semantics=("arbitrary",),
            vmem_limit_bytes=32 * 1024 * 1024),
    )(u_bf, i_bf)

    nblk = Nip // 256
    s_i32 = jax.lax.bitcast_convert_type(
        smat.reshape(Nu, Nip // 2, 2), jnp.int32)
    s_tab = s_i32.reshape(Nu * nblk, 1, 128)

    Bp = pl.cdiv(B, _TILE) * _TILE
    pad = Bp - B
    uid = jnp.pad(user_ids.astype(jnp.int32), (0, pad))
    iid = jnp.pad(item_ids.astype(jnp.int32), (0, pad))
    fb = (uid * nblk + (iid >> 8)).reshape(1, Bp)
    nch = Bp // _CHUNK
    ilo3 = jnp.pad(
        (iid & 255).reshape(nch, _CHUNK // 8, 8).transpose(0, 2, 1),
        ((0, 0), (0, 0), (0, 128 - _CHUNK // 8)))

    out = pl.pallas_call(
        _gather_kernel,
        out_shape=jax.ShapeDtypeStruct((1, Bp), jnp.float32),
        grid=(Bp // _TILE,),
        in_specs=[
            pl.BlockSpec((1, _TILE), lambda i: (0, i),
                         memory_space=pltpu.SMEM),
            pl.BlockSpec((_TILE // _CHUNK, 8, 128), lambda i: (i, 0, 0)),
            pl.BlockSpec((Nu * nblk, 1, 128), lambda i: (0, 0, 0)),
        ],
        out_specs=pl.BlockSpec((1, _TILE), lambda i: (0, i)),
        compiler_params=pltpu.CompilerParams(
            dimension_semantics=("parallel",),
            vmem_limit_bytes=56 * 1024 * 1024),
    )(fb, ilo3, s_tab)

    return out[0, :B]


# speedup vs baseline: 1.6239x; 1.6239x over previous
import jax
import jax.numpy as jnp
from jax.experimental import pallas as pl
from jax.experimental.pallas import tpu as pltpu

_TILE = 32768
_UNROLL = 256
_GRP = 256


def _score_matrix_kernel(u_ref, i_ref, out_ref):
    out_ref[...] = jax.lax.dot_general(
        u_ref[...], i_ref[...], (((1,), (1,)), ((), ())),
        preferred_element_type=jnp.float32).astype(jnp.bfloat16)


def _gather_kernel(fb_ref, ilo_ref, s_ref, out_ref, p_ref):
    def chunk(ci, carry):
        base = ci * _UNROLL
        rows = []
        for j in range(_UNROLL):
            rows.append(s_ref[fb_ref[0, base + j]])
        blk = jnp.concatenate(rows, axis=0)
        off = pl.multiple_of(ci * _UNROLL, _UNROLL)
        p_ref[pl.ds(off, _UNROLL), :] = blk
        return carry

    jax.lax.fori_loop(0, _TILE // _UNROLL, chunk, 0)

    ones = jnp.ones((8, 128), jnp.bfloat16)
    iota = jax.lax.broadcasted_iota(jnp.int32, (_GRP, 128), 1)
    ncol = _GRP // 8
    for g in range(_TILE // _GRP):
        x = p_ref[g * _GRP:(g + 1) * _GRP, :]
        ev = pltpu.bitcast(x << 16, jnp.float32)
        od = pltpu.bitcast(x & jnp.int32(-65536), jnp.float32)
        l0f = jnp.concatenate(
            [jnp.broadcast_to(ilo_ref[:, v:v + 1], (8, 128))
             for v in range(g * ncol, (g + 1) * ncol)], axis=0)
        sel = jnp.where((l0f & 1) == 1, od, ev)
        blk2 = jnp.where(iota == (l0f >> 1), sel, 0.0).astype(jnp.bfloat16)
        acc = jax.lax.dot_general(ones, blk2, (((1,), (1,)), ((), ())),
                                  preferred_element_type=jnp.float32)
        out_ref[:, g * _GRP:(g + 1) * _GRP] = acc[0:1, :]


def kernel(user_ids, item_ids, user_emb, item_emb, user_bias, item_bias):
    B = user_ids.shape[0]
    Nu, D = user_emb.shape
    Ni = item_emb.shape[0]
    Daug = D + 2

    u_aug = jnp.concatenate(
        [user_emb.astype(jnp.float32),
         user_bias.reshape(Nu, 1).astype(jnp.float32),
         jnp.ones((Nu, 1), jnp.float32)], axis=1)
    i_aug = jnp.concatenate(
        [item_emb.astype(jnp.float32),
         jnp.ones((Ni, 1), jnp.float32),
         item_bias.reshape(Ni, 1).astype(jnp.float32)], axis=1)

    Dpad = ((Daug + 127) // 128) * 128
    Nip = ((Ni + 255) // 256) * 256
    u_aug = jnp.pad(u_aug, ((0, 0), (0, Dpad - Daug)))
    i_aug = jnp.pad(i_aug, ((0, Nip - Ni), (0, Dpad - Daug)))
    u_bf = u_aug.astype(jnp.bfloat16)
    i_bf = i_aug.astype(jnp.bfloat16)

    ublk = min(512, Nu)
    smat = pl.pallas_call(
        _score_matrix_kernel,
        out_shape=jax.ShapeDtypeStruct((Nu, Nip), jnp.bfloat16),
        grid=(Nu // ublk,),
        in_specs=[
            pl.BlockSpec((ublk, Dpad), lambda i: (i, 0)),
            pl.BlockSpec((Nip, Dpad), lambda i: (0, 0)),
        ],
        out_specs=pl.BlockSpec((ublk, Nip), lambda i: (i, 0)),
        compiler_params=pltpu.CompilerParams(
            dimension_semantics=("arbitrary",),
            vmem_limit_bytes=32 * 1024 * 1024),
    )(u_bf, i_bf)

    nblk = Nip // 256
    s_i32 = jax.lax.bitcast_convert_type(
        smat.reshape(Nu, Nip // 2, 2), jnp.int32)
    s_tab = s_i32.reshape(Nu * nblk, 1, 128)

    Bp = pl.cdiv(B, _TILE) * _TILE
    pad = Bp - B
    uid = jnp.pad(user_ids.astype(jnp.int32), (0, pad))
    iid = jnp.pad(item_ids.astype(jnp.int32), (0, pad))
    fb = (uid * nblk + (iid >> 8)).reshape(1, Bp)
    ilo_col = (iid & 255).reshape(Bp // 8, 8).T

    out = pl.pallas_call(
        _gather_kernel,
        out_shape=jax.ShapeDtypeStruct((1, Bp), jnp.float32),
        grid=(Bp // _TILE,),
        in_specs=[
            pl.BlockSpec((1, _TILE), lambda i: (0, i),
                         memory_space=pltpu.SMEM),
            pl.BlockSpec((8, _TILE // 8), lambda i: (0, i)),
            pl.BlockSpec((Nu * nblk, 1, 128), lambda i: (0, 0, 0)),
        ],
        out_specs=pl.BlockSpec((1, _TILE), lambda i: (0, i)),
        scratch_shapes=[pltpu.VMEM((_TILE, 128), jnp.int32)],
        compiler_params=pltpu.CompilerParams(
            dimension_semantics=("parallel",),
            vmem_limit_bytes=56 * 1024 * 1024),
    )(fb, ilo_col, s_tab)

    return out[0, :B]


# speedup vs baseline: 1.6266x; 1.0017x over previous
import jax
import jax.numpy as jnp
from jax.experimental import pallas as pl
from jax.experimental.pallas import tpu as pltpu

_TILE = 32768
_UNROLL = 256
_GRP = 256


def _score_matrix_kernel(u_ref, i_ref, out_ref):
    out_ref[...] = jax.lax.dot_general(
        u_ref[...], i_ref[...], (((1,), (1,)), ((), ())),
        preferred_element_type=jnp.float32).astype(jnp.bfloat16)


def _gather_kernel(fb_ref, ilo_ref, s_ref, out_ref, p_ref):
    def chunk(ci, carry):
        base = ci * _UNROLL
        rows = []
        for j in range(_UNROLL):
            rows.append(s_ref[fb_ref[0, base + j]])
        blk = jnp.concatenate(rows, axis=0)
        off = pl.multiple_of(ci * _UNROLL, _UNROLL)
        p_ref[pl.ds(off, _UNROLL), :] = blk
        return carry

    jax.lax.fori_loop(0, _TILE // _UNROLL, chunk, 0)

    ones = jnp.ones((8, 128), jnp.bfloat16)
    iota = jax.lax.broadcasted_iota(jnp.int32, (_GRP, 128), 1)
    ncol = _GRP // 8
    for g in range(_TILE // _GRP):
        x = p_ref[g * _GRP:(g + 1) * _GRP, :]
        ev = pltpu.bitcast(x << 16, jnp.float32)
        od = pltpu.bitcast(x & jnp.int32(-65536), jnp.float32)
        l0f = jnp.concatenate(
            [jnp.broadcast_to(ilo_ref[:, v:v + 1], (8, 128))
             for v in range(g * ncol, (g + 1) * ncol)], axis=0)
        sel = jnp.where((l0f & 1) == 1, od, ev)
        blk2 = jnp.where(iota == (l0f >> 1), sel, 0.0).astype(jnp.bfloat16)
        acc = jax.lax.dot_general(ones, blk2, (((1,), (1,)), ((), ())),
                                  preferred_element_type=jnp.float32)
        out_ref[:, g * _GRP:(g + 1) * _GRP] = acc[0:1, :]


def kernel(user_ids, item_ids, user_emb, item_emb, user_bias, item_bias):
    B = user_ids.shape[0]
    Nu, D = user_emb.shape
    Ni = item_emb.shape[0]
    Daug = D + 2

    u_aug = jnp.concatenate(
        [user_emb.astype(jnp.float32),
         user_bias.reshape(Nu, 1).astype(jnp.float32),
         jnp.ones((Nu, 1), jnp.float32)], axis=1)
    i_aug = jnp.concatenate(
        [item_emb.astype(jnp.float32),
         jnp.ones((Ni, 1), jnp.float32),
         item_bias.reshape(Ni, 1).astype(jnp.float32)], axis=1)

    Dpad = ((Daug + 127) // 128) * 128
    Nip = ((Ni + 255) // 256) * 256
    u_aug = jnp.pad(u_aug, ((0, 0), (0, Dpad - Daug)))
    i_aug = jnp.pad(i_aug, ((0, Nip - Ni), (0, Dpad - Daug)))
    u_bf = u_aug.astype(jnp.bfloat16)
    i_bf = i_aug.astype(jnp.bfloat16)

    ublk = min(512, Nu)
    smat = pl.pallas_call(
        _score_matrix_kernel,
        out_shape=jax.ShapeDtypeStruct((Nu, Nip), jnp.bfloat16),
        grid=(Nu // ublk,),
        in_specs=[
            pl.BlockSpec((ublk, Dpad), lambda i: (i, 0)),
            pl.BlockSpec((Nip, Dpad), lambda i: (0, 0)),
        ],
        out_specs=pl.BlockSpec((ublk, Nip), lambda i: (i, 0)),
        compiler_params=pltpu.CompilerParams(
            dimension_semantics=("arbitrary",),
            vmem_limit_bytes=32 * 1024 * 1024),
    )(u_bf, i_bf)

    nblk = Nip // 256
    s_i32 = jax.lax.bitcast_convert_type(
        smat.reshape(Nu, Nip // 2, 2), jnp.int32)
    s_tab = s_i32.reshape(Nu * nblk, 1, 128)

    Bp = pl.cdiv(B, _TILE) * _TILE
    pad = Bp - B
    uid = jnp.pad(user_ids.astype(jnp.int32), (0, pad))
    iid = jnp.pad(item_ids.astype(jnp.int32), (0, pad))
    fb = (uid * nblk + (iid >> 8)).reshape(1, Bp)
    ilo_col = (iid & 255).reshape(Bp // 8, 8).T

    out = pl.pallas_call(
        _gather_kernel,
        out_shape=jax.ShapeDtypeStruct((1, Bp), jnp.float32),
        grid=(Bp // _TILE,),
        in_specs=[
            pl.BlockSpec((1, _TILE), lambda i: (0, i),
                         memory_space=pltpu.SMEM),
            pl.BlockSpec((8, _TILE // 8), lambda i: (0, i)),
            pl.BlockSpec((Nu * nblk, 1, 128), lambda i: (0, 0, 0)),
        ],
        out_specs=pl.BlockSpec((1, _TILE), lambda i: (0, i)),
        scratch_shapes=[pltpu.VMEM((_TILE, 128), jnp.int32)],
        compiler_params=pltpu.CompilerParams(
            dimension_semantics=("parallel",),
            disable_bounds_checks=True,
            vmem_limit_bytes=56 * 1024 * 1024),
    )(fb, ilo_col, s_tab)

    return out[0, :B]
